# baseline (speedup 1.0000x reference)
import functools
import jax
import jax.numpy as jnp
from jax import lax
from jax.experimental import pallas as pl
from jax.experimental.pallas import tpu as pltpu
from jax.experimental.pallas import tpu_sc as plsc


def _make_sc_kernel(B, maxlen, D):
    info = plsc.get_sparse_core_info()
    NC, NS, L = info.num_cores, info.num_subcores, info.num_lanes
    NW = NC * NS
    C = maxlen // 2
    n_chunks = (B * maxlen) // (NW * C)
    n_pairs = n_chunks // 2
    n_tg = C // L
    HALF = D // 2
    mesh = plsc.VectorSubcoreMesh(core_axis_name="c", subcore_axis_name="s")

    @functools.partial(
        pl.kernel,
        mesh=mesh,
        compiler_params=pltpu.CompilerParams(
            use_tc_tiling_on_sc=False, needs_layout_passes=False
        ),
        out_type=jax.ShapeDtypeStruct((B, D, maxlen), jnp.float32),
        scratch_types=[
            pltpu.VMEM((C,), jnp.int32),
            pltpu.VMEM((C,), jnp.int32),
            pltpu.VMEM((D, maxlen), jnp.float32),
            pltpu.VMEM((C, D), jnp.float32),
            pltpu.VMEM((C, D), jnp.float32),
            pltpu.VMEM((D, C), jnp.float32),
            pltpu.SemaphoreType.DMA,
            pltpu.SemaphoreType.DMA,
            pltpu.SemaphoreType.DMA,
            pltpu.SemaphoreType.DMA,
        ],
    )
    def k(x_hbm, tok_hbm, post_hbm, out_hbm,
          idx0, idx1, pos_v, rows0, rows1, plane, semg0, semg1, sems0, sems1):
        wid = lax.axis_index("s") * NC + lax.axis_index("c")
        first = wid * n_chunks
        pltpu.sync_copy(post_hbm, pos_v)

        row_iota = jax.lax.iota(jnp.int32, L)
        store_sems = (sems0, sems1)

        def compute_store(rows, chunk, phase, skip_store_wait):
            batch = chunk // 2
            for h in range(2):
                half_plane = plane.at[pl.ds(h * HALF, HALF), :]
                dst = out_hbm.at[
                    batch, pl.ds(h * HALF, HALF), pl.ds(phase * C, C)
                ]
                desc = pltpu.make_async_copy(half_plane, dst, store_sems[h])
                if skip_store_wait is None:
                    desc.wait()
                else:
                    @pl.when(jnp.logical_not(skip_store_wait))
                    def _():
                        pltpu.make_async_copy(
                            half_plane, dst, store_sems[h]
                        ).wait()

                def f_body(f, carry):
                    cvec = jnp.full((L,), f, jnp.int32)

                    @plsc.parallel_loop(0, n_tg, 1, unroll=2)
                    def tg_loop(tg):
                        ridx = row_iota + tg * L
                        vals = plsc.load_gather(rows, [ridx, cvec])
                        pvec = pos_v[f, pl.ds(phase * C + tg * L, L)]
                        plane[f, pl.ds(tg * L, L)] = vals + pvec

                    return carry

                lax.fori_loop(h * HALF, (h + 1) * HALF, f_body, 0)

                pltpu.async_copy(half_plane, dst, store_sems[h])

        pltpu.sync_copy(x_hbm.at[pl.ds(first * C, C)], idx0)
        pltpu.async_copy(tok_hbm.at[idx0], rows0, semg0)

        def pair_body(j, carry):
            ca = first + 2 * j
            pltpu.sync_copy(x_hbm.at[pl.ds((ca + 1) * C, C)], idx1)
            pltpu.async_copy(tok_hbm.at[idx1], rows1, semg1)
            pltpu.make_async_copy(tok_hbm.at[idx0], rows0, semg0).wait()
            compute_store(rows0, ca, 0, skip_store_wait=(j == 0))
            @pl.when(j + 1 < n_pairs)
            def _():
                pltpu.sync_copy(x_hbm.at[pl.ds((ca + 2) * C, C)], idx0)
                pltpu.async_copy(tok_hbm.at[idx0], rows0, semg0)
            pltpu.make_async_copy(tok_hbm.at[idx1], rows1, semg1).wait()
            compute_store(rows1, ca + 1, 1, skip_store_wait=None)
            return carry

        lax.fori_loop(0, n_pairs, pair_body, 0)

        last_b = first // 2 + n_chunks // 2 - 1
        for h in range(2):
            pltpu.make_async_copy(
                plane.at[pl.ds(h * HALF, HALF), :],
                out_hbm.at[last_b, pl.ds(h * HALF, HALF), pl.ds(C, C)],
                store_sems[h],
            ).wait()

    return k


def kernel(x, token_table, pos_table):
    B, maxlen = x.shape
    V, D = token_table.shape
    x_flat = x.reshape(-1).astype(jnp.int32)
    pos_t = pos_table.T
    k = _make_sc_kernel(B, maxlen, D)
    out_t = k(x_flat, token_table, pos_t)
    return out_t.transpose(0, 2, 1)


# speedup vs baseline: 1.3288x; 1.3288x over previous
import functools
import jax
import jax.numpy as jnp
from jax import lax
from jax.experimental import pallas as pl
from jax.experimental.pallas import tpu as pltpu
from jax.experimental.pallas import tpu_sc as plsc


def _make_sc_kernel(B, maxlen, D, W):
    info = plsc.get_sparse_core_info()
    NC, NS, L = info.num_cores, info.num_subcores, info.num_lanes
    NW = NC * NS
    C = 128
    n_phases = maxlen // C
    n_chunks = (B * maxlen) // (NW * C)
    n_pairs = n_chunks // 2
    n_tg = C // L
    mesh = plsc.VectorSubcoreMesh(core_axis_name="c", subcore_axis_name="s")

    @functools.partial(
        pl.kernel,
        mesh=mesh,
        compiler_params=pltpu.CompilerParams(
            use_tc_tiling_on_sc=True, needs_layout_passes=False
        ),
        out_type=jax.ShapeDtypeStruct((B, D, maxlen), jnp.float32),
        scratch_types=[
            pltpu.VMEM((C,), jnp.int32),
            pltpu.VMEM((C,), jnp.int32),
            pltpu.VMEM((n_phases, D, C), jnp.float32),
            pltpu.VMEM((C, W), jnp.float32),
            pltpu.VMEM((C, W), jnp.float32),
            pltpu.VMEM((D, C), jnp.float32),
            pltpu.VMEM((D, C), jnp.float32),
            pltpu.SemaphoreType.DMA,
            pltpu.SemaphoreType.DMA,
            pltpu.SemaphoreType.DMA,
            pltpu.SemaphoreType.DMA,
        ],
    )
    def k(x_hbm, tok_hbm, pos_hbm, out_hbm,
          idx0, idx1, pos_v, rows0, rows1, plane0, plane1,
          semg0, semg1, sems0, sems1):
        wid = lax.axis_index("s") * NC + lax.axis_index("c")
        first = wid * n_chunks
        pltpu.sync_copy(pos_hbm, pos_v)

        row_iota = jax.lax.iota(jnp.int32, L)

        def compute_store(rows, plane, sem, chunk):
            batch = chunk // n_phases
            phase = lax.rem(chunk, n_phases)
            dst = out_hbm.at[batch, :, pl.ds(phase * C, C)]

            @pl.when(chunk >= first + 2)
            def _():
                pltpu.make_async_copy(plane, dst, sem).wait()

            @plsc.parallel_loop(0, D, 1, unroll=2)
            def init_loop(f):
                for tg in range(n_tg):
                    plane[f, pl.ds(tg * L, L)] = pos_v[
                        phase, f, pl.ds(tg * L, L)
                    ]

            @plsc.parallel_loop(0, D, 1, unroll=2)
            def tr_loop(f):
                cvec = jnp.full((L,), f, jnp.int32)
                for tg in range(n_tg):
                    vals = plsc.load_gather(rows, [row_iota + tg * L, cvec])
                    plsc.addupdate(plane.at[f, pl.ds(tg * L, L)], vals)

            pltpu.async_copy(plane, dst, sem)

        pltpu.sync_copy(x_hbm.at[pl.ds(first * C, C)], idx0)
        pltpu.async_copy(tok_hbm.at[idx0], rows0, semg0)

        def pair_body(j, carry):
            ca = first + 2 * j
            pltpu.sync_copy(x_hbm.at[pl.ds((ca + 1) * C, C)], idx1)
            pltpu.async_copy(tok_hbm.at[idx1], rows1, semg1)
            pltpu.make_async_copy(tok_hbm.at[idx0], rows0, semg0).wait()
            compute_store(rows0, plane0, sems0, ca)
            @pl.when(j + 1 < n_pairs)
            def _():
                pltpu.sync_copy(x_hbm.at[pl.ds((ca + 2) * C, C)], idx0)
                pltpu.async_copy(tok_hbm.at[idx0], rows0, semg0)
            pltpu.make_async_copy(tok_hbm.at[idx1], rows1, semg1).wait()
            compute_store(rows1, plane1, sems1, ca + 1)
            return carry

        lax.fori_loop(0, n_pairs, pair_body, 0)

        last = first + n_chunks - 1
        for plane, sem, chunk in ((plane0, sems0, last - 1), (plane1, sems1, last)):
            batch = chunk // n_phases
            phase = chunk % n_phases
            pltpu.make_async_copy(
                plane, out_hbm.at[batch, :, pl.ds(phase * C, C)], sem
            ).wait()

    return k


def kernel(x, token_table, pos_table):
    B, maxlen = x.shape
    V, D = token_table.shape
    W = 2 * D
    x_flat = x.reshape(-1).astype(jnp.int32)
    tok_p = jnp.pad(token_table, ((0, 0), (0, W - D)))
    pos_p = pos_table.T.reshape(D, maxlen // 128, 128).swapaxes(0, 1)
    k = _make_sc_kernel(B, maxlen, D, W)
    out_t = k(x_flat, tok_p, pos_p)
    return out_t.transpose(0, 2, 1)


# speedup vs baseline: 1.4255x; 1.0727x over previous
import functools
import jax
import jax.numpy as jnp
from jax import lax
from jax.experimental import pallas as pl
from jax.experimental.pallas import tpu as pltpu
from jax.experimental.pallas import tpu_sc as plsc


def _make_sc_kernel(B, maxlen, D, W):
    info = plsc.get_sparse_core_info()
    NC, NS, L = info.num_cores, info.num_subcores, info.num_lanes
    NW = NC * NS
    C = 128
    n_phases = maxlen // C
    n_chunks = (B * maxlen) // (NW * C)
    n_pairs = n_chunks // 2
    n_tg = C // L
    mesh = plsc.VectorSubcoreMesh(core_axis_name="c", subcore_axis_name="s")

    @functools.partial(
        pl.kernel,
        mesh=mesh,
        compiler_params=pltpu.CompilerParams(
            use_tc_tiling_on_sc=True, needs_layout_passes=False
        ),
        out_type=jax.ShapeDtypeStruct((B, D, maxlen), jnp.float32),
        scratch_types=[
            pltpu.VMEM((192 * C,), jnp.int32),
            pltpu.VMEM((n_phases, D, C), jnp.float32),
            pltpu.VMEM((C, W), jnp.float32),
            pltpu.VMEM((C, W), jnp.float32),
            pltpu.VMEM((D, C), jnp.float32),
            pltpu.VMEM((D, C), jnp.float32),
            pltpu.SemaphoreType.DMA,
            pltpu.SemaphoreType.DMA,
            pltpu.SemaphoreType.DMA,
            pltpu.SemaphoreType.DMA,
        ],
    )
    def k(x_hbm, tok_hbm, pos_hbm, out_hbm,
          idx_all, pos_v, rows0, rows1, plane0, plane1,
          semg0, semg1, sems0, sems1):
        wid = lax.axis_index("s") * NC + lax.axis_index("c")
        first = wid * n_chunks
        pltpu.sync_copy(pos_hbm, pos_v)

        row_iota = jax.lax.iota(jnp.int32, L)

        def compute_store(rows, plane, sem, chunk):
            batch = chunk // n_phases
            phase = lax.rem(chunk, n_phases)
            dst = out_hbm.at[batch, :, pl.ds(phase * C, C)]

            @pl.when(chunk >= first + 2)
            def _():
                pltpu.make_async_copy(plane, dst, sem).wait()

            @plsc.parallel_loop(0, D, 1, unroll=2)
            def init_loop(f):
                for tg in range(n_tg):
                    plane[f, pl.ds(tg * L, L)] = pos_v[
                        phase, f, pl.ds(tg * L, L)
                    ]

            @plsc.parallel_loop(0, D, 1, unroll=2)
            def tr_loop(f):
                cvec = jnp.full((L,), f, jnp.int32)
                for tg in range(n_tg):
                    vals = plsc.load_gather(rows, [row_iota + tg * L, cvec])
                    plsc.addupdate(plane.at[f, pl.ds(tg * L, L)], vals)

            pltpu.async_copy(plane, dst, sem)

        pltpu.sync_copy(x_hbm.at[pl.ds(first * C, n_chunks * C)], idx_all)

        def gidx(c_local):
            return idx_all.at[pl.ds(c_local * C, C)]

        pltpu.async_copy(tok_hbm.at[gidx(0)], rows0, semg0)

        def pair_body(j, carry):
            ca = first + 2 * j
            pltpu.async_copy(tok_hbm.at[gidx(2 * j + 1)], rows1, semg1)
            pltpu.make_async_copy(tok_hbm.at[gidx(2 * j)], rows0, semg0).wait()
            compute_store(rows0, plane0, sems0, ca)
            @pl.when(j + 1 < n_pairs)
            def _():
                pltpu.async_copy(tok_hbm.at[gidx(2 * j + 2)], rows0, semg0)
            pltpu.make_async_copy(tok_hbm.at[gidx(2 * j + 1)], rows1, semg1).wait()
            compute_store(rows1, plane1, sems1, ca + 1)
            return carry

        lax.fori_loop(0, n_pairs, pair_body, 0)

        last = first + n_chunks - 1
        for plane, sem, chunk in ((plane0, sems0, last - 1), (plane1, sems1, last)):
            batch = chunk // n_phases
            phase = chunk % n_phases
            pltpu.make_async_copy(
                plane, out_hbm.at[batch, :, pl.ds(phase * C, C)], sem
            ).wait()

    return k


def kernel(x, token_table, pos_table):
    B, maxlen = x.shape
    V, D = token_table.shape
    W = 2 * D
    x_flat = x.reshape(-1).astype(jnp.int32)
    tok_p = jnp.pad(token_table, ((0, 0), (0, W - D)))
    pos_p = pos_table.T.reshape(D, maxlen // 128, 128).swapaxes(0, 1)
    k = _make_sc_kernel(B, maxlen, D, W)
    out_t = k(x_flat, tok_p, pos_p)
    return out_t.transpose(0, 2, 1)
